# Initial kernel scaffold; baseline (speedup 1.0000x reference)
import jax
import jax.numpy as jnp
from jax.experimental import pallas as pl


def kernel(category, operator_class, rest_features, cat_emb, op_emb):
    raise NotImplementedError("write your pallas kernel here")


# speedup vs baseline: 1.4875x; 1.4875x over previous
import functools

import jax
import jax.numpy as jnp
from jax import lax
from jax.experimental import pallas as pl
from jax.experimental.pallas import tpu as pltpu
from jax.experimental.pallas import tpu_sc as plsc

_G = 128


@functools.partial(jax.jit, static_argnames=())
def _encode(cat_idx3, op_idx3, rest_features, cat_emb, op_emb):
    info = plsc.get_sparse_core_info()
    nw = info.num_cores * info.num_subcores
    n, d_rest = rest_features.shape
    d_cat = cat_emb.shape[1]
    d_op = op_emb.shape[1]
    d_out = d_cat + d_op + d_rest
    per_w = n // nw
    assert per_w * nw == n
    ngrp = op_idx3.shape[1]
    n_full = per_w // _G
    tail = per_w - n_full * _G

    mesh = plsc.VectorSubcoreMesh(core_axis_name="c", subcore_axis_name="s")

    @functools.partial(
        pl.kernel,
        mesh=mesh,
        compiler_params=pltpu.CompilerParams(use_tc_tiling_on_sc=False),
        out_type=jax.ShapeDtypeStruct((n, d_out), jnp.float32),
        scratch_types=[
            pltpu.VMEM((ngrp, _G), jnp.int32),
            pltpu.VMEM((ngrp, _G), jnp.int32),
            pltpu.VMEM((_G, d_cat), jnp.float32),
            pltpu.VMEM((_G, d_op), jnp.float32),
            pltpu.VMEM((_G, d_rest), jnp.float32),
            pltpu.SemaphoreType.DMA,
            pltpu.SemaphoreType.DMA,
            pltpu.SemaphoreType.DMA,
        ],
    )
    def k(cat_idx_hbm, op_idx_hbm, rest_hbm, cat_emb_hbm, op_emb_hbm,
          out_hbm, idxc, idxo, catbuf, opbuf, restbuf, sem1, sem2, sem3):
        wid = lax.axis_index("s") * info.num_cores + lax.axis_index("c")
        row_base = wid * per_w

        pltpu.sync_copy(cat_idx_hbm.at[wid], idxc)
        pltpu.sync_copy(op_idx_hbm.at[wid], idxo)

        def do_group(j, nrows):
            row0 = row_base + j * _G
            a = pltpu.async_copy(cat_emb_hbm.at[idxc.at[j]], catbuf, sem1)
            b = pltpu.async_copy(op_emb_hbm.at[idxo.at[j]], opbuf, sem2)
            c = pltpu.async_copy(
                rest_hbm.at[pl.ds(row0, nrows)],
                restbuf.at[pl.ds(0, nrows)], sem3)
            a.wait()
            b.wait()
            c.wait()
            pltpu.sync_copy(
                catbuf.at[pl.ds(0, nrows)],
                out_hbm.at[pl.ds(row0, nrows), pl.ds(0, d_cat)])
            pltpu.sync_copy(
                opbuf.at[pl.ds(0, nrows)],
                out_hbm.at[pl.ds(row0, nrows), pl.ds(d_cat, d_op)])
            pltpu.sync_copy(
                restbuf.at[pl.ds(0, nrows)],
                out_hbm.at[pl.ds(row0, nrows), pl.ds(d_cat + d_op, d_rest)])

        def body(j, carry):
            do_group(j, _G)
            return carry

        lax.fori_loop(0, n_full, body, 0)
        if tail:
            do_group(n_full, tail)

    return k(cat_idx3, op_idx3, rest_features, cat_emb, op_emb)


def kernel(category, operator_class, rest_features, cat_emb, op_emb):
    info = plsc.get_sparse_core_info()
    nw = info.num_cores * info.num_subcores
    n = category.shape[0]
    per_w = n // nw
    assert per_w * nw == n, "row count must divide evenly across subcores"
    ngrp = -(-per_w // _G)
    pos = (jnp.arange(nw, dtype=jnp.int32)[:, None] * per_w
           + jnp.arange(ngrp * _G, dtype=jnp.int32)[None, :])
    pos = jnp.minimum(pos, n - 1).reshape(nw, ngrp, _G)
    cat_idx3 = jnp.take(category.astype(jnp.int32), pos, mode="clip")
    op_idx3 = jnp.take(operator_class.astype(jnp.int32), pos, mode="clip")
    return _encode(cat_idx3, op_idx3, rest_features, cat_emb, op_emb)
